# baseline (speedup 1.0000x reference)
import dataclasses
import jax
import jax.numpy as jnp
from jax import lax
from jax.experimental import pallas as pl
from jax.experimental.pallas import tpu as pltpu
from jax.experimental.pallas import tpu_sc as plsc

N_NODES = 10000
N_EDGES = 320000
D = 128
DW = D // 2
NC = 2
NS = 16
WPT = DW // NS
E_PER_C = N_EDGES // NC
CH = 1280
N_CH = E_PER_C // CH
R_PER_W = N_NODES // NS
R_BLK = 25
N_RBLK = R_PER_W // R_BLK


def _dot_kernel(model_hbm, edge_hbm, out_hbm,
                packed_sh, slice_v, sidx_v, didx_v, pbuf_v,
                pin_v, pout_v, isem0, isem1, osem):
  cid = lax.axis_index("c")
  sid = lax.axis_index("s")
  ebase = cid * E_PER_C

  def issue_idx(c, p):
    pltpu.async_copy(edge_hbm.at[0, pl.ds(ebase + c * CH, CH)],
                     sidx_v.at[p], isem0.at[p])
    pltpu.async_copy(edge_hbm.at[1, pl.ds(ebase + c * CH, CH)],
                     didx_v.at[p], isem1.at[p])

  def wait_idx(p):
    pltpu.make_async_copy(edge_hbm.at[0, pl.ds(0, CH)], sidx_v.at[p],
                          isem0.at[p]).wait()
    pltpu.make_async_copy(edge_hbm.at[1, pl.ds(0, CH)], didx_v.at[p],
                          isem1.at[p]).wait()

  issue_idx(0, 0)
  issue_idx(1, 1)

  for h in (0, 1):
    @pl.loop(0, N_RBLK)
    def _pack(blk):
      row0 = sid * R_PER_W + blk * R_BLK
      pltpu.sync_copy(
          model_hbm.at[pl.ds(row0, R_BLK), pl.ds(64 * h, 64)], pin_v)

      @pl.loop(0, R_BLK)
      def _row(r):
        for k in range(2):
          a = pin_v[r, pl.ds(32 * k, 16)]
          b = pin_v[r, pl.ds(32 * k + 16, 16)]
          p = plsc.pack(a, b, format=plsc.PackFormat.INTERLEAVED)
          pout_v[r, pl.ds(16 * k, 16)] = plsc.bitcast(p, jnp.int32)

      pltpu.sync_copy(pout_v, packed_sh.at[pl.ds(row0, R_BLK)])

    plsc.subcore_barrier()

    @pl.when(sid // 8 == h)
    def _():
      pltpu.sync_copy(
          packed_sh.at[pl.ds(0, N_NODES), pl.ds(WPT * sid - 32 * h, WPT)],
          slice_v)

    plsc.subcore_barrier()

  wvecs = [jnp.full((16,), w, jnp.int32) for w in range(WPT)]

  def compute(p):
    @pl.loop(0, CH // 16)
    def _grp(g):
      sidx16 = sidx_v[p, pl.ds(g * 16, 16)]
      didx16 = didx_v[p, pl.ds(g * 16, 16)]
      prods = []
      for w in range(WPT):
        s_w = plsc.load_gather(slice_v, [sidx16, wvecs[w]])
        d_w = plsc.load_gather(slice_v, [didx16, wvecs[w]])
        prods.append(plsc.bitcast(s_w, jnp.bfloat16) *
                     plsc.bitcast(d_w, jnp.bfloat16))
      acc_bf = (prods[0] + prods[1]) + (prods[2] + prods[3])
      u0, u1 = plsc.unpack(acc_bf, format=plsc.PackFormat.INTERLEAVED)
      pbuf_v[p, pl.ds(g * 16, 16)] = u0 + u1

  @pl.loop(0, N_CH - 1, step=2)
  def _main(c):
    for p in (0, 1):
      wait_idx(p)
      compute(p)

      @pl.when(c + p + 2 < N_CH)
      def _():
        issue_idx(c + p + 2, p)

  wait_idx(0)
  compute(0)

  @pl.when(sid == 0)
  def _():
    pltpu.sync_copy(pbuf_v.at[0], out_hbm.at[cid, pl.ds(0, CH)])
    pltpu.sync_copy(pbuf_v.at[1], out_hbm.at[cid, pl.ds(CH, CH)])


@jax.jit
def kernel(model, edge_index):
  edge_index = edge_index.astype(jnp.int32)
  mesh = plsc.VectorSubcoreMesh(core_axis_name="c", subcore_axis_name="s")
  cp = pltpu.CompilerParams()
  if "needs_layout_passes" in pltpu.CompilerParams.__dataclass_fields__:
    cp = dataclasses.replace(cp, needs_layout_passes=False)
  cp = dataclasses.replace(cp, use_tc_tiling_on_sc=False)
  k = pl.kernel(
      _dot_kernel,
      out_type=jax.ShapeDtypeStruct((NC, E_PER_C), jnp.float32),
      mesh=mesh,
      scratch_types=[
          pltpu.VMEM_SHARED((N_NODES, DW // 2), jnp.int32),
          pltpu.VMEM((N_NODES, WPT), jnp.int32),
          pltpu.VMEM((2, CH), jnp.int32),
          pltpu.VMEM((2, CH), jnp.int32),
          pltpu.VMEM((2, CH), jnp.float32),
          pltpu.VMEM((R_BLK, D // 2), jnp.float32),
          pltpu.VMEM((R_BLK, DW // 2), jnp.int32),
          pltpu.SemaphoreType.DMA((2,)),
          pltpu.SemaphoreType.DMA((2,)),
          pltpu.SemaphoreType.DMA((2,)),
      ],
      compiler_params=cp,
  )
  return k(model, edge_index).reshape(N_EDGES)


# speedup vs baseline: 2.7279x; 2.7279x over previous
import dataclasses
import jax
import jax.numpy as jnp
from jax import lax
from jax.experimental import pallas as pl
from jax.experimental.pallas import tpu as pltpu
from jax.experimental.pallas import tpu_sc as plsc

N_NODES = 10000
N_EDGES = 320000
D = 128
DW = D // 2
NC = 2
NS = 16
NW = NC * NS
E_PER_W = N_EDGES // NW
CHUNK = 80
N_CHUNKS = E_PER_W // CHUNK
NBUF = 4
R_PER_W = N_NODES // NS
R_BLK = 25
N_RBLK = R_PER_W // R_BLK


def _dot_kernel(model_hbm, edge_hbm, out_hbm,
                packed_sh, sidx_v, didx_v, rows0_v, rows1_v, out_v,
                pin_v, pout_v, gsem0, gsem1, osem):
  cid = lax.axis_index("c")
  sid = lax.axis_index("s")
  wid = sid * NC + cid
  ebase = wid * E_PER_W

  icp0 = pltpu.async_copy(edge_hbm.at[0, pl.ds(ebase, E_PER_W)], sidx_v,
                          gsem0.at[0])
  icp1 = pltpu.async_copy(edge_hbm.at[1, pl.ds(ebase, E_PER_W)], didx_v,
                          gsem1.at[0])

  def prow(blk):
    return sid * R_PER_W + blk * R_BLK

  def pack_in(blk, ph):
    return pltpu.async_copy(model_hbm.at[pl.ds(prow(blk), R_BLK)],
                            pin_v.at[ph], osem.at[ph])

  def pack_out(blk, ph):
    return pltpu.async_copy(pout_v.at[ph],
                            packed_sh.at[pl.ds(prow(blk), R_BLK)],
                            osem.at[2 + ph])

  pack_in(0, 0)
  pack_in(1, 1)
  for blk in range(N_RBLK):
    ph = blk % 2
    pltpu.make_async_copy(model_hbm.at[pl.ds(prow(blk), R_BLK)],
                          pin_v.at[ph], osem.at[ph]).wait()
    if blk >= 2:
      pltpu.make_async_copy(pout_v.at[ph],
                            packed_sh.at[pl.ds(prow(blk), R_BLK)],
                            osem.at[2 + ph]).wait()

    @pl.loop(0, R_BLK)
    def _row(r):
      for k in range(D // 32):
        a = pin_v[ph, r, pl.ds(32 * k, 16)]
        b = pin_v[ph, r, pl.ds(32 * k + 16, 16)]
        p = plsc.pack(a, b, format=plsc.PackFormat.INTERLEAVED)
        pout_v[ph, r, pl.ds(16 * k, 16)] = plsc.bitcast(p, jnp.int32)

    pack_out(blk, ph)
    if blk + 2 < N_RBLK:
      pack_in(blk + 2, ph)

  for blk in (N_RBLK - 2, N_RBLK - 1):
    ph = blk % 2
    pltpu.make_async_copy(pout_v.at[ph],
                          packed_sh.at[pl.ds(prow(blk), R_BLK)],
                          osem.at[2 + ph]).wait()

  icp0.wait()
  icp1.wait()
  plsc.subcore_barrier()

  def issue_gather(chunk, b):
    table = packed_sh
    s_idx = sidx_v.at[pl.ds(chunk * CHUNK, CHUNK)]
    d_idx = didx_v.at[pl.ds(chunk * CHUNK, CHUNK)]
    pltpu.async_copy(table.at[s_idx], rows0_v.at[b], gsem0.at[b])
    pltpu.async_copy(table.at[d_idx], rows1_v.at[b], gsem1.at[b])

  def wait_gather(b):
    table = packed_sh
    s_idx = sidx_v.at[pl.ds(0, CHUNK)]
    d_idx = didx_v.at[pl.ds(0, CHUNK)]
    pltpu.make_async_copy(table.at[s_idx], rows0_v.at[b], gsem0.at[b]).wait()
    pltpu.make_async_copy(table.at[d_idx], rows1_v.at[b], gsem1.at[b]).wait()

  def out_store_wait(chunk, b):
    pltpu.make_async_copy(
        out_v.at[b], out_hbm.at[pl.ds(ebase + chunk * CHUNK, CHUNK)],
        osem.at[b]).wait()

  def compute(chunk, b):
    @pl.loop(0, CHUNK // 16)
    def _grp(g):
      outv = jnp.zeros((16,), jnp.float32)
      for j in range(16):
        e = g * 16 + j
        prods = []
        for k in range(D // 32):
          s = plsc.bitcast(rows0_v[b, e, pl.ds(16 * k, 16)], jnp.bfloat16)
          d = plsc.bitcast(rows1_v[b, e, pl.ds(16 * k, 16)], jnp.bfloat16)
          prods.append(s * d)
        acc_bf = (prods[0] + prods[1]) + (prods[2] + prods[3])
        u0, u1 = plsc.unpack(acc_bf, format=plsc.PackFormat.INTERLEAVED)
        mask = lax.iota(jnp.int32, 16) == j
        outv = jnp.where(mask, jnp.sum(u0 + u1), outv)
      out_v[b, pl.ds(g * 16, 16)] = outv

  issue_gather(0, 0)
  issue_gather(1, 1)
  issue_gather(2, 2)

  @pl.loop(0, N_CHUNKS - 1, step=NBUF)
  def _ring(c):
    for b in range(NBUF):
      chunk = c + b
      wait_gather(b)

      @pl.when(chunk + (NBUF - 1) <= N_CHUNKS - 1)
      def _():
        issue_gather(chunk + (NBUF - 1), (b + NBUF - 1) % NBUF)

      @pl.when(chunk >= NBUF)
      def _():
        out_store_wait(chunk - NBUF, b)

      compute(chunk, b)
      pltpu.async_copy(
          out_v.at[b], out_hbm.at[pl.ds(ebase + chunk * CHUNK, CHUNK)],
          osem.at[b])

  last = N_CHUNKS - 1
  wait_gather(0)
  out_store_wait(last - NBUF, 0)
  compute(last, 0)
  pltpu.sync_copy(out_v.at[0],
                  out_hbm.at[pl.ds(ebase + last * CHUNK, CHUNK)])
  out_store_wait(last - 3, 1)
  out_store_wait(last - 2, 2)
  out_store_wait(last - 1, 3)


@jax.jit
def kernel(model, edge_index):
  edge_index = edge_index.astype(jnp.int32)
  mesh = plsc.VectorSubcoreMesh(core_axis_name="c", subcore_axis_name="s")
  cp = pltpu.CompilerParams()
  if "needs_layout_passes" in pltpu.CompilerParams.__dataclass_fields__:
    cp = dataclasses.replace(cp, needs_layout_passes=False)
  cp = dataclasses.replace(cp, use_tc_tiling_on_sc=False)
  k = pl.kernel(
      _dot_kernel,
      out_type=jax.ShapeDtypeStruct((N_EDGES,), jnp.float32),
      mesh=mesh,
      scratch_types=[
          pltpu.VMEM_SHARED((N_NODES, DW), jnp.int32),
          pltpu.VMEM((E_PER_W,), jnp.int32),
          pltpu.VMEM((E_PER_W,), jnp.int32),
          pltpu.VMEM((NBUF, CHUNK, DW), jnp.int32),
          pltpu.VMEM((NBUF, CHUNK, DW), jnp.int32),
          pltpu.VMEM((NBUF, CHUNK), jnp.float32),
          pltpu.VMEM((2, R_BLK, D), jnp.float32),
          pltpu.VMEM((2, R_BLK, DW), jnp.int32),
          pltpu.SemaphoreType.DMA((NBUF,)),
          pltpu.SemaphoreType.DMA((NBUF,)),
          pltpu.SemaphoreType.DMA((NBUF,)),
      ],
      compiler_params=cp,
  )
  return k(model, edge_index)
